# baseline (speedup 1.0000x reference)
import jax
import jax.numpy as jnp
from jax.experimental import pallas as pl
from jax.experimental.pallas import tpu as pltpu

_BLOCK = 2 * 1024 * 1024


def _eta_body(eta_ref, mask_ref, x_ref, o_ref):
    e = eta_ref[...]
    m = mask_ref[...]
    x = x_ref[...]
    o_ref[...] = jnp.where(x > e, x * (m * e), 1.0 - x / e)


def kernel(loss, eta, mask):
    n = loss.shape[0]
    out = pl.pallas_call(
        _eta_body,
        grid=(n // _BLOCK,),
        in_specs=[
            pl.BlockSpec((1,), lambda i: (0,)),
            pl.BlockSpec((1,), lambda i: (0,)),
            pl.BlockSpec((_BLOCK,), lambda i: (i,)),
        ],
        out_specs=pl.BlockSpec((_BLOCK,), lambda i: (i,)),
        out_shape=jax.ShapeDtypeStruct((n,), jnp.float32),
        compiler_params=pltpu.CompilerParams(
            dimension_semantics=("parallel",),
            vmem_limit_bytes=48 * 1024 * 1024,
        ),
    )(eta, mask, loss)
    return out


# speedup vs baseline: 1.0232x; 1.0232x over previous
import jax
import jax.numpy as jnp
from jax.experimental import pallas as pl
from jax.experimental.pallas import tpu as pltpu

_BLOCK = 2 * 1024 * 1024


def _eta_body(x_ref, eta_ref, mask_ref, o_ref):
    e = eta_ref[0]
    m = mask_ref[0]
    x = x_ref[...]
    o_ref[...] = jnp.where(x > e, x * (m * e), 1.0 - x / e)


def kernel(loss, eta, mask):
    n = loss.shape[0]
    out = pl.pallas_call(
        _eta_body,
        grid=(n // _BLOCK,),
        in_specs=[
            pl.BlockSpec((_BLOCK,), lambda i: (i,)),
            pl.BlockSpec(memory_space=pltpu.SMEM),
            pl.BlockSpec(memory_space=pltpu.SMEM),
        ],
        out_specs=pl.BlockSpec((_BLOCK,), lambda i: (i,)),
        out_shape=jax.ShapeDtypeStruct((n,), jnp.float32),
        compiler_params=pltpu.CompilerParams(
            dimension_semantics=("parallel",),
            vmem_limit_bytes=48 * 1024 * 1024,
        ),
    )(loss, eta, mask)
    return out


# speedup vs baseline: 1.0236x; 1.0004x over previous
import jax
import jax.numpy as jnp
from jax.experimental import pallas as pl
from jax.experimental.pallas import tpu as pltpu

_BLOCK = 2 * 1024 * 1024


def _eta_body(eta_ref, mask_ref, x_ref, o_ref):
    e = eta_ref[0]
    m = mask_ref[0]
    x = x_ref[...]
    o_ref[...] = jnp.where(x > e, x * (m * e), 1.0 - x / e)


def kernel(loss, eta, mask):
    n = loss.shape[0]
    out = pl.pallas_call(
        _eta_body,
        grid_spec=pltpu.PrefetchScalarGridSpec(
            num_scalar_prefetch=2,
            grid=(n // _BLOCK,),
            in_specs=[pl.BlockSpec((_BLOCK,), lambda i, e, m: (i,))],
            out_specs=pl.BlockSpec((_BLOCK,), lambda i, e, m: (i,)),
        ),
        out_shape=jax.ShapeDtypeStruct((n,), jnp.float32),
        compiler_params=pltpu.CompilerParams(
            dimension_semantics=("parallel",),
            vmem_limit_bytes=48 * 1024 * 1024,
        ),
    )(eta, mask, loss)
    return out
